# baseline (speedup 1.0000x reference)
import dataclasses
import functools

import jax
import jax.numpy as jnp
from jax import lax
from jax.experimental import pallas as pl
from jax.experimental.pallas import tpu as pltpu
from jax.experimental.pallas import tpu_sc as plsc

N = 10000
NPAD = 10112
E = 320000
BLK = 128
NW = 32
NB0 = 80
NB1 = 80
NBMX = max(NB0, NB1)
NBS = NBMX + 2
E_PAD = 16 * (NB0 + NB1) * BLK
RPS = NPAD // 16
BM = 1000


def _vec_mesh():
    return plsc.VectorSubcoreMesh(core_axis_name="c", subcore_axis_name="s")


def _sc_params():
    cp = pltpu.CompilerParams()
    fields = pltpu.CompilerParams.__dataclass_fields__
    if "needs_layout_passes" in fields:
        cp = dataclasses.replace(cp, needs_layout_passes=False)
    if "use_tc_tiling_on_sc" in fields:
        cp = dataclasses.replace(cp, use_tc_tiling_on_sc=False)
    return cp


def _deg_sc(dst3, w3):
    @functools.partial(
        pl.kernel,
        out_type=jax.ShapeDtypeStruct((2, NPAD), jnp.float32),
        mesh=_vec_mesh(),
        compiler_params=_sc_params(),
        scratch_types=[
            pltpu.VMEM((NBS, BLK), jnp.int32),
            pltpu.VMEM((NBS, BLK), jnp.float32),
            pltpu.VMEM_SHARED((NPAD,), jnp.float32),
            pltpu.SemaphoreType.DMA,
        ],
    )
    def k(dst_hbm, w_hbm, out_hbm, dst_v, w_v, acc_sh, sem):
        cid = lax.axis_index("c")
        sid = lax.axis_index("s")
        nb = jnp.where(cid == 0, NB0, NB1)
        pltpu.sync_copy(dst_hbm.at[cid, sid], dst_v)
        pltpu.sync_copy(w_hbm.at[cid, sid], w_v)
        base = sid * RPS
        for off, cnt in ((0, 128), (128, 128), (256, 128), (384, 128),
                         (512, 120)):
            pltpu.sync_copy(w_v.at[NBS - 1, pl.ds(0, cnt)],
                            acc_sh.at[pl.ds(base + off, cnt)])
        plsc.subcore_barrier()

        @pl.loop(0, nb)
        def _(b):
            pltpu.async_copy(w_v.at[b], acc_sh.at[dst_v.at[b]], sem,
                             add=True)

        @pl.loop(0, nb)
        def _(b):
            pltpu.make_async_copy(w_v.at[0], acc_sh.at[dst_v.at[0]],
                                  sem).wait()

        plsc.subcore_barrier()
        pltpu.sync_copy(acc_sh.at[pl.ds(sid * RPS, RPS)],
                        out_hbm.at[cid, pl.ds(sid * RPS, RPS)])

    return k(dst3, w3)


def _agg_sc(src3, dst3, w3, table, d):
    @functools.partial(
        pl.kernel,
        out_type=jax.ShapeDtypeStruct((2, NPAD, d), jnp.float32),
        mesh=_vec_mesh(),
        compiler_params=_sc_params(),
        scratch_types=[
            pltpu.VMEM((BLK,), jnp.int32),
            pltpu.VMEM((BLK,), jnp.int32),
            pltpu.VMEM((BLK,), jnp.int32),
            pltpu.VMEM((BLK,), jnp.int32),
            pltpu.VMEM((BLK,), jnp.int32),
            pltpu.VMEM((BLK,), jnp.int32),
            pltpu.VMEM((BLK,), jnp.float32),
            pltpu.VMEM((BLK,), jnp.float32),
            pltpu.VMEM((BLK, d), jnp.float32),
            pltpu.VMEM((BLK, d), jnp.float32),
            pltpu.VMEM_SHARED((NPAD, d), jnp.float32),
            pltpu.SemaphoreType.DMA,
            pltpu.SemaphoreType.DMA,
            pltpu.SemaphoreType.DMA,
            pltpu.SemaphoreType.DMA,
            pltpu.SemaphoreType.DMA,
            pltpu.SemaphoreType.DMA,
        ],
    )
    def k(src_hbm, dst_hbm, w_hbm, tab_hbm, out_hbm,
          s0, s1, d0, d1, d2, d3, w0, w1, r0, r1, acc_sh,
          sem_i0, sem_i1, sem_g0, sem_g1, sem_s0, sem_s1):
        cid = lax.axis_index("c")
        sid = lax.axis_index("s")
        nb = jnp.where(cid == 0, NB0, NB1)
        sbuf = [s0, s1]
        dbuf = [d0, d1, d2, d3]
        wbuf = [w0, w1]
        rbuf = [r0, r1]
        sem_i = [sem_i0, sem_i1]
        sem_g = [sem_g0, sem_g1]
        sem_s = [sem_s0, sem_s1]

        @plsc.parallel_loop(0, BLK, unroll=4)
        def _(e):
            for c in range(d // 16):
                r0[e, pl.ds(c * 16, 16)] = jnp.zeros((16,), jnp.float32)

        base = sid * RPS
        for off, cnt in ((0, 128), (128, 128), (256, 128), (384, 128),
                         (512, 120)):
            pltpu.async_copy(r0.at[pl.ds(0, cnt)],
                             acc_sh.at[pl.ds(base + off, cnt)], sem_g0)
        for off, cnt in ((0, 128), (128, 128), (256, 128), (384, 128),
                         (512, 120)):
            pltpu.make_async_copy(r0.at[pl.ds(0, cnt)],
                                  acc_sh.at[pl.ds(base + off, cnt)],
                                  sem_g0).wait()
        plsc.subcore_barrier()

        pltpu.sync_copy(src_hbm.at[cid, sid, 0], s0)
        pltpu.sync_copy(w_hbm.at[cid, sid, 0], w0)
        pltpu.sync_copy(dst_hbm.at[cid, sid, 0], d0)
        pltpu.async_copy(tab_hbm.at[s0], r0, sem_g0)
        pltpu.async_copy(src_hbm.at[cid, sid, 1], s1, sem_i1)
        pltpu.async_copy(w_hbm.at[cid, sid, 1], w1, sem_i1)
        pltpu.async_copy(dst_hbm.at[cid, sid, 1], d1, sem_i1)

        def scale(rows, wv):
            @plsc.parallel_loop(0, BLK, unroll=4)
            def _(e):
                ws = plsc.load_gather(wv, [jnp.full((16,), e, jnp.int32)])
                for c in range(d // 16):
                    sl = pl.ds(c * 16, 16)
                    rows[e, sl] = rows[e, sl] * ws

        def half(t, k_, b):
            p = k_ % 2
            q = 1 - p
            pltpu.make_async_copy(src_hbm.at[cid, sid, 0], sbuf[q],
                                  sem_i[q]).wait()
            pltpu.make_async_copy(w_hbm.at[cid, sid, 0], wbuf[q],
                                  sem_i[q]).wait()
            pltpu.make_async_copy(dst_hbm.at[cid, sid, 0], dbuf[(k_ + 1) % 4],
                                  sem_i[q]).wait()

            @pl.when(b >= 1)
            def _():
                pltpu.make_async_copy(rbuf[q], acc_sh.at[dbuf[(k_ + 3) % 4]],
                                      sem_s[q]).wait()

            pltpu.async_copy(tab_hbm.at[sbuf[q]], rbuf[q], sem_g[q])
            pltpu.make_async_copy(tab_hbm.at[pl.ds(0, BLK)], rbuf[p],
                                  sem_g[p]).wait()
            scale(rbuf[p], wbuf[p])
            pltpu.async_copy(rbuf[p], acc_sh.at[dbuf[k_]], sem_s[p], add=True)
            pltpu.async_copy(src_hbm.at[cid, sid, b + 2], sbuf[p], sem_i[p])
            pltpu.async_copy(w_hbm.at[cid, sid, b + 2], wbuf[p], sem_i[p])
            pltpu.async_copy(dst_hbm.at[cid, sid, b + 2], dbuf[(k_ + 2) % 4],
                             sem_i[p])

        @pl.loop(0, nb // 4)
        def _(t):
            for k_ in range(4):
                half(t, k_, t * 4 + k_)

        pltpu.make_async_copy(rbuf[1], acc_sh.at[dbuf[3]], sem_s[1]).wait()
        pltpu.make_async_copy(tab_hbm.at[pl.ds(0, BLK)], rbuf[0],
                              sem_g[0]).wait()
        pltpu.make_async_copy(src_hbm.at[cid, sid, 0], sbuf[1],
                              sem_i[1]).wait()
        pltpu.make_async_copy(w_hbm.at[cid, sid, 0], wbuf[1],
                              sem_i[1]).wait()
        pltpu.make_async_copy(dst_hbm.at[cid, sid, 0], dbuf[3],
                              sem_i[1]).wait()

        plsc.subcore_barrier()
        pltpu.sync_copy(acc_sh.at[pl.ds(sid * RPS, RPS)],
                        out_hbm.at[cid, pl.ds(sid * RPS, RPS)])

    return k(src3, dst3, w3, table)


def _dis_block(p0_ref, p1_ref):
    deg = p0_ref[...] + p1_ref[...] + 1.0
    return lax.rsqrt(deg)


def _matmul_tc(x, wp):
    m, kdim = x.shape
    n = wp.shape[1]

    def body(x_ref, w_ref, o_ref):
        o_ref[...] = jnp.dot(x_ref[...], w_ref[...],
                             preferred_element_type=jnp.float32,
                             precision=lax.Precision.HIGHEST)

    return pl.pallas_call(
        body,
        grid=(m // BM,),
        in_specs=[pl.BlockSpec((BM, kdim), lambda i: (i, 0)),
                  pl.BlockSpec((kdim, n), lambda i: (0, 0))],
        out_specs=pl.BlockSpec((BM, n), lambda i: (i, 0)),
        out_shape=jax.ShapeDtypeStruct((m, n), jnp.float32),
    )(x, wp)


def _scale_tc(p0, p1, h):
    m, n = h.shape

    def body(p0_ref, p1_ref, h_ref, o_ref):
        o_ref[...] = h_ref[...] * _dis_block(p0_ref, p1_ref)

    return pl.pallas_call(
        body,
        grid=(m // BM,),
        in_specs=[pl.BlockSpec((BM, 1), lambda i: (i, 0)),
                  pl.BlockSpec((BM, 1), lambda i: (i, 0)),
                  pl.BlockSpec((BM, n), lambda i: (i, 0))],
        out_specs=pl.BlockSpec((BM, n), lambda i: (i, 0)),
        out_shape=jax.ShapeDtypeStruct((m, n), jnp.float32),
    )(p0, p1, h)


def _layer_tc(p0, p1, sa, sb, g, bp, w2p):
    m, n = g.shape
    n2 = w2p.shape[1]

    def body(p0_ref, p1_ref, sa_ref, sb_ref, g_ref, b_ref, w2_ref, o_ref):
        dis = _dis_block(p0_ref, p1_ref)
        t = dis * (sa_ref[...] + sb_ref[...] + g_ref[...]) + b_ref[...]
        t = jnp.maximum(t, 0.0)
        h2 = jnp.dot(t, w2_ref[...], preferred_element_type=jnp.float32,
                     precision=lax.Precision.HIGHEST)
        o_ref[...] = dis * h2

    return pl.pallas_call(
        body,
        grid=(m // BM,),
        in_specs=[pl.BlockSpec((BM, 1), lambda i: (i, 0)),
                  pl.BlockSpec((BM, 1), lambda i: (i, 0)),
                  pl.BlockSpec((BM, n), lambda i: (i, 0)),
                  pl.BlockSpec((BM, n), lambda i: (i, 0)),
                  pl.BlockSpec((BM, n), lambda i: (i, 0)),
                  pl.BlockSpec((1, n), lambda i: (0, 0)),
                  pl.BlockSpec((n, n2), lambda i: (0, 0))],
        out_specs=pl.BlockSpec((BM, n2), lambda i: (i, 0)),
        out_shape=jax.ShapeDtypeStruct((m, n2), jnp.float32),
    )(p0, p1, sa, sb, g, bp, w2p)


def _final_tc(p0, p1, sa, sb, g, bp):
    m, n = g.shape

    def body(p0_ref, p1_ref, sa_ref, sb_ref, g_ref, b_ref, o_ref):
        dis = _dis_block(p0_ref, p1_ref)
        t = dis * (sa_ref[...] + sb_ref[...] + g_ref[...]) + b_ref[...]
        o_ref[...] = jnp.maximum(t, 0.0)

    return pl.pallas_call(
        body,
        grid=(m // BM,),
        in_specs=[pl.BlockSpec((BM, 1), lambda i: (i, 0)),
                  pl.BlockSpec((BM, 1), lambda i: (i, 0)),
                  pl.BlockSpec((BM, n), lambda i: (i, 0)),
                  pl.BlockSpec((BM, n), lambda i: (i, 0)),
                  pl.BlockSpec((BM, n), lambda i: (i, 0)),
                  pl.BlockSpec((1, n), lambda i: (0, 0))],
        out_specs=pl.BlockSpec((BM, n), lambda i: (i, 0)),
        out_shape=jax.ShapeDtypeStruct((m, n), jnp.float32),
    )(p0, p1, sa, sb, g, bp)


def kernel(x, edge_index, edge_attr, W1, b1, W2, b2):
    src = edge_index[0].astype(jnp.int32)
    dst = edge_index[1].astype(jnp.int32)
    w = edge_attr.astype(jnp.float32)
    pad = E_PAD - E
    c0 = 16 * NB0 * BLK

    def to4(a):
        af = jnp.pad(a, (0, pad))
        p0 = af[:c0].reshape(16, NB0, BLK)
        p0 = jnp.concatenate(
            [p0, jnp.zeros((16, NBS - NB0, BLK), a.dtype)], axis=1)
        p1 = af[c0:].reshape(16, NB1, BLK)
        p1 = jnp.concatenate(
            [p1, jnp.zeros((16, NBS - NB1, BLK), a.dtype)], axis=1)
        return jnp.stack([p0, p1])

    src3 = to4(src)
    dst3 = to4(dst)
    w3 = to4(w)

    w1p = jnp.pad(W1, ((0, 0), (0, 3)))
    b1p = jnp.pad(b1, (0, 3)).reshape(1, 128)
    w2p = jnp.pad(W2, ((0, 3), (0, 7)))
    b2p = jnp.pad(b2, (0, 7)).reshape(1, 32)

    degp = _deg_sc(dst3, w3)
    h1 = _matmul_tc(x, w1p)
    p0 = degp[0, :N].reshape(N, 1)
    p1 = degp[1, :N].reshape(N, 1)
    g1 = _scale_tc(p0, p1, h1)

    s1 = _agg_sc(src3, dst3, w3, g1, 128)
    g2 = _layer_tc(p0, p1, s1[0, :N], s1[1, :N], g1, b1p, w2p)

    s2 = _agg_sc(src3, dst3, w3, g2, 32)
    out = _final_tc(p0, p1, s2[0, :N], s2[1, :N], g2, b2p)
    return out[:, :25]


# speedup vs baseline: 1.2650x; 1.2650x over previous
import dataclasses
import functools

import jax
import jax.numpy as jnp
from jax import lax
from jax.experimental import pallas as pl
from jax.experimental.pallas import tpu as pltpu
from jax.experimental.pallas import tpu_sc as plsc

N = 10000
NPAD = 10112
E = 320000
BLK = 128
NW = 32
NB0 = 104
NB1 = 56
NBMX = max(NB0, NB1)
NBS = NBMX + 2
E_PAD = 16 * (NB0 + NB1) * BLK
RPS = NPAD // 16
BM = 1000


def _vec_mesh():
    return plsc.VectorSubcoreMesh(core_axis_name="c", subcore_axis_name="s")


def _sc_params():
    cp = pltpu.CompilerParams()
    fields = pltpu.CompilerParams.__dataclass_fields__
    if "needs_layout_passes" in fields:
        cp = dataclasses.replace(cp, needs_layout_passes=False)
    if "use_tc_tiling_on_sc" in fields:
        cp = dataclasses.replace(cp, use_tc_tiling_on_sc=False)
    return cp


def _deg_sc(dst3, w3):
    @functools.partial(
        pl.kernel,
        out_type=jax.ShapeDtypeStruct((2, NPAD), jnp.float32),
        mesh=_vec_mesh(),
        compiler_params=_sc_params(),
        scratch_types=[
            pltpu.VMEM((NBS, BLK), jnp.int32),
            pltpu.VMEM((NBS, BLK), jnp.float32),
            pltpu.VMEM_SHARED((NPAD,), jnp.float32),
            pltpu.SemaphoreType.DMA,
        ],
    )
    def k(dst_hbm, w_hbm, out_hbm, dst_v, w_v, acc_sh, sem):
        cid = lax.axis_index("c")
        sid = lax.axis_index("s")
        nb = jnp.where(cid == 0, NB0, NB1)
        pltpu.sync_copy(dst_hbm.at[cid, sid], dst_v)
        pltpu.sync_copy(w_hbm.at[cid, sid], w_v)
        base = sid * RPS
        for off, cnt in ((0, 128), (128, 128), (256, 128), (384, 128),
                         (512, 120)):
            pltpu.sync_copy(w_v.at[NBS - 1, pl.ds(0, cnt)],
                            acc_sh.at[pl.ds(base + off, cnt)])
        plsc.subcore_barrier()

        @pl.loop(0, nb)
        def _(b):
            pltpu.async_copy(w_v.at[b], acc_sh.at[dst_v.at[b]], sem,
                             add=True)

        @pl.loop(0, nb)
        def _(b):
            pltpu.make_async_copy(w_v.at[0], acc_sh.at[dst_v.at[0]],
                                  sem).wait()

        plsc.subcore_barrier()
        pltpu.sync_copy(acc_sh.at[pl.ds(sid * RPS, RPS)],
                        out_hbm.at[cid, pl.ds(sid * RPS, RPS)])

    return k(dst3, w3)


def _agg_sc(src3, dst3, w3, table, zeros, d):
    @functools.partial(
        pl.kernel,
        out_type=jax.ShapeDtypeStruct((2, NPAD, d), jnp.float32),
        mesh=_vec_mesh(),
        compiler_params=_sc_params(),
        scratch_types=[
            pltpu.VMEM((BLK,), jnp.int32),
            pltpu.VMEM((BLK,), jnp.int32),
            pltpu.VMEM((BLK,), jnp.int32),
            pltpu.VMEM((BLK,), jnp.int32),
            pltpu.VMEM((BLK,), jnp.int32),
            pltpu.VMEM((BLK,), jnp.int32),
            pltpu.VMEM((BLK,), jnp.float32),
            pltpu.VMEM((BLK,), jnp.float32),
            pltpu.VMEM((BLK, d), jnp.float32),
            pltpu.VMEM((BLK, d), jnp.float32),
            pltpu.VMEM_SHARED((NPAD, d), jnp.float32),
            pltpu.SemaphoreType.DMA,
            pltpu.SemaphoreType.DMA,
            pltpu.SemaphoreType.DMA,
            pltpu.SemaphoreType.DMA,
            pltpu.SemaphoreType.DMA,
            pltpu.SemaphoreType.DMA,
        ],
    )
    def k(src_hbm, dst_hbm, w_hbm, tab_hbm, z_hbm, out_hbm,
          s0, s1, d0, d1, d2, d3, w0, w1, r0, r1, acc_sh,
          sem_i0, sem_i1, sem_g0, sem_g1, sem_s0, sem_s1):
        cid = lax.axis_index("c")
        sid = lax.axis_index("s")
        nb = jnp.where(cid == 0, NB0, NB1)
        sbuf = [s0, s1]
        dbuf = [d0, d1, d2, d3]
        wbuf = [w0, w1]
        rbuf = [r0, r1]
        sem_i = [sem_i0, sem_i1]
        sem_g = [sem_g0, sem_g1]
        sem_s = [sem_s0, sem_s1]

        pltpu.sync_copy(z_hbm, acc_sh.at[pl.ds(sid * RPS, RPS)])
        plsc.subcore_barrier()

        pltpu.sync_copy(src_hbm.at[cid, sid, 0], s0)
        pltpu.sync_copy(w_hbm.at[cid, sid, 0], w0)
        pltpu.sync_copy(dst_hbm.at[cid, sid, 0], d0)
        pltpu.async_copy(tab_hbm.at[s0], r0, sem_g0)
        pltpu.async_copy(src_hbm.at[cid, sid, 1], s1, sem_i1)
        pltpu.async_copy(w_hbm.at[cid, sid, 1], w1, sem_i1)
        pltpu.async_copy(dst_hbm.at[cid, sid, 1], d1, sem_i1)

        def scale(rows, wv):
            @plsc.parallel_loop(0, BLK, unroll=4)
            def _(e):
                ws = plsc.load_gather(wv, [jnp.full((16,), e, jnp.int32)])
                for c in range(d // 16):
                    sl = pl.ds(c * 16, 16)
                    rows[e, sl] = rows[e, sl] * ws

        def half(t, k_, b):
            p = k_ % 2
            q = 1 - p
            pltpu.make_async_copy(src_hbm.at[cid, sid, 0], sbuf[q],
                                  sem_i[q]).wait()
            pltpu.make_async_copy(w_hbm.at[cid, sid, 0], wbuf[q],
                                  sem_i[q]).wait()
            pltpu.make_async_copy(dst_hbm.at[cid, sid, 0], dbuf[(k_ + 1) % 4],
                                  sem_i[q]).wait()

            @pl.when(b >= 1)
            def _():
                pltpu.make_async_copy(rbuf[q], acc_sh.at[dbuf[(k_ + 3) % 4]],
                                      sem_s[q]).wait()

            pltpu.async_copy(tab_hbm.at[sbuf[q]], rbuf[q], sem_g[q])
            pltpu.make_async_copy(tab_hbm.at[pl.ds(0, BLK)], rbuf[p],
                                  sem_g[p]).wait()
            scale(rbuf[p], wbuf[p])
            pltpu.async_copy(rbuf[p], acc_sh.at[dbuf[k_]], sem_s[p], add=True)
            pltpu.async_copy(src_hbm.at[cid, sid, b + 2], sbuf[p], sem_i[p])
            pltpu.async_copy(w_hbm.at[cid, sid, b + 2], wbuf[p], sem_i[p])
            pltpu.async_copy(dst_hbm.at[cid, sid, b + 2], dbuf[(k_ + 2) % 4],
                             sem_i[p])

        @pl.loop(0, nb // 4)
        def _(t):
            for k_ in range(4):
                half(t, k_, t * 4 + k_)

        pltpu.make_async_copy(rbuf[1], acc_sh.at[dbuf[3]], sem_s[1]).wait()
        pltpu.make_async_copy(tab_hbm.at[pl.ds(0, BLK)], rbuf[0],
                              sem_g[0]).wait()
        pltpu.make_async_copy(src_hbm.at[cid, sid, 0], sbuf[1],
                              sem_i[1]).wait()
        pltpu.make_async_copy(w_hbm.at[cid, sid, 0], wbuf[1],
                              sem_i[1]).wait()
        pltpu.make_async_copy(dst_hbm.at[cid, sid, 0], dbuf[3],
                              sem_i[1]).wait()

        plsc.subcore_barrier()
        pltpu.sync_copy(acc_sh.at[pl.ds(sid * RPS, RPS)],
                        out_hbm.at[cid, pl.ds(sid * RPS, RPS)])

    return k(src3, dst3, w3, table, zeros)


def _dis_block(p0_ref, p1_ref):
    deg = p0_ref[...] + p1_ref[...] + 1.0
    return lax.rsqrt(deg)


def _matmul_tc(x, wp):
    m, kdim = x.shape
    n = wp.shape[1]

    def body(x_ref, w_ref, o_ref):
        o_ref[...] = jnp.dot(x_ref[...], w_ref[...],
                             preferred_element_type=jnp.float32,
                             precision=lax.Precision.HIGHEST)

    return pl.pallas_call(
        body,
        grid=(m // BM,),
        in_specs=[pl.BlockSpec((BM, kdim), lambda i: (i, 0)),
                  pl.BlockSpec((kdim, n), lambda i: (0, 0))],
        out_specs=pl.BlockSpec((BM, n), lambda i: (i, 0)),
        out_shape=jax.ShapeDtypeStruct((m, n), jnp.float32),
    )(x, wp)


def _scale_tc(p0, p1, h):
    m, n = h.shape

    def body(p0_ref, p1_ref, h_ref, o_ref):
        o_ref[...] = h_ref[...] * _dis_block(p0_ref, p1_ref)

    return pl.pallas_call(
        body,
        grid=(m // BM,),
        in_specs=[pl.BlockSpec((BM, 1), lambda i: (i, 0)),
                  pl.BlockSpec((BM, 1), lambda i: (i, 0)),
                  pl.BlockSpec((BM, n), lambda i: (i, 0))],
        out_specs=pl.BlockSpec((BM, n), lambda i: (i, 0)),
        out_shape=jax.ShapeDtypeStruct((m, n), jnp.float32),
    )(p0, p1, h)


def _layer_tc(p0, p1, sa, sb, g, bp, w2p):
    m, n = g.shape
    n2 = w2p.shape[1]

    def body(p0_ref, p1_ref, sa_ref, sb_ref, g_ref, b_ref, w2_ref, o_ref):
        dis = _dis_block(p0_ref, p1_ref)
        t = dis * (sa_ref[...] + sb_ref[...] + g_ref[...]) + b_ref[...]
        t = jnp.maximum(t, 0.0)
        h2 = jnp.dot(t, w2_ref[...], preferred_element_type=jnp.float32,
                     precision=lax.Precision.HIGHEST)
        o_ref[...] = dis * h2

    return pl.pallas_call(
        body,
        grid=(m // BM,),
        in_specs=[pl.BlockSpec((BM, 1), lambda i: (i, 0)),
                  pl.BlockSpec((BM, 1), lambda i: (i, 0)),
                  pl.BlockSpec((BM, n), lambda i: (i, 0)),
                  pl.BlockSpec((BM, n), lambda i: (i, 0)),
                  pl.BlockSpec((BM, n), lambda i: (i, 0)),
                  pl.BlockSpec((1, n), lambda i: (0, 0)),
                  pl.BlockSpec((n, n2), lambda i: (0, 0))],
        out_specs=pl.BlockSpec((BM, n2), lambda i: (i, 0)),
        out_shape=jax.ShapeDtypeStruct((m, n2), jnp.float32),
    )(p0, p1, sa, sb, g, bp, w2p)


def _final_tc(p0, p1, sa, sb, g, bp):
    m, n = g.shape

    def body(p0_ref, p1_ref, sa_ref, sb_ref, g_ref, b_ref, o_ref):
        dis = _dis_block(p0_ref, p1_ref)
        t = dis * (sa_ref[...] + sb_ref[...] + g_ref[...]) + b_ref[...]
        o_ref[...] = jnp.maximum(t, 0.0)

    return pl.pallas_call(
        body,
        grid=(m // BM,),
        in_specs=[pl.BlockSpec((BM, 1), lambda i: (i, 0)),
                  pl.BlockSpec((BM, 1), lambda i: (i, 0)),
                  pl.BlockSpec((BM, n), lambda i: (i, 0)),
                  pl.BlockSpec((BM, n), lambda i: (i, 0)),
                  pl.BlockSpec((BM, n), lambda i: (i, 0)),
                  pl.BlockSpec((1, n), lambda i: (0, 0))],
        out_specs=pl.BlockSpec((BM, n), lambda i: (i, 0)),
        out_shape=jax.ShapeDtypeStruct((m, n), jnp.float32),
    )(p0, p1, sa, sb, g, bp)


def kernel(x, edge_index, edge_attr, W1, b1, W2, b2):
    src = edge_index[0].astype(jnp.int32)
    dst = edge_index[1].astype(jnp.int32)
    w = edge_attr.astype(jnp.float32)
    pad = E_PAD - E
    c0 = 16 * NB0 * BLK

    def to4(a):
        af = jnp.pad(a, (0, pad))
        p0 = af[:c0].reshape(16, NB0, BLK)
        p0 = jnp.concatenate(
            [p0, jnp.zeros((16, NBS - NB0, BLK), a.dtype)], axis=1)
        p1 = af[c0:].reshape(16, NB1, BLK)
        p1 = jnp.concatenate(
            [p1, jnp.zeros((16, NBS - NB1, BLK), a.dtype)], axis=1)
        return jnp.stack([p0, p1])

    src3 = to4(src)
    dst3 = to4(dst)
    w3 = to4(w)

    w1p = jnp.pad(W1, ((0, 0), (0, 3)))
    b1p = jnp.pad(b1, (0, 3)).reshape(1, 128)
    w2p = jnp.pad(W2, ((0, 3), (0, 7)))
    b2p = jnp.pad(b2, (0, 7)).reshape(1, 32)

    z128 = jnp.zeros((RPS, 128), jnp.float32)
    z32 = jnp.zeros((RPS, 32), jnp.float32)

    degp = _deg_sc(dst3, w3)
    h1 = _matmul_tc(x, w1p)
    p0 = degp[0, :N].reshape(N, 1)
    p1 = degp[1, :N].reshape(N, 1)
    g1 = _scale_tc(p0, p1, h1)

    s1 = _agg_sc(src3, dst3, w3, g1, z128, 128)
    g2 = _layer_tc(p0, p1, s1[0, :N], s1[1, :N], g1, b1p, w2p)

    s2 = _agg_sc(src3, dst3, w3, g2, z32, 32)
    out = _final_tc(p0, p1, s2[0, :N], s2[1, :N], g2, b2p)
    return out[:, :25]
